# Initial kernel scaffold; baseline (speedup 1.0000x reference)
import jax
import jax.numpy as jnp
from jax.experimental import pallas as pl


def kernel(x, edge_index, batch, roi_scaler, W1, b1, W2, b2, gate_W, gate_b, p1_W, p1_b, bn_gamma, bn_beta, p2_W, p2_b):
    raise NotImplementedError("write your pallas kernel here")


# speedup vs baseline: 8.7301x; 8.7301x over previous
import functools

import jax
import jax.numpy as jnp
from jax import lax
from jax.experimental import pallas as pl
from jax.experimental.pallas import tpu as pltpu
from jax.experimental.pallas import tpu_sc as plsc

_CHUNK = 128


def _sc_mesh():
    return plsc.VectorSubcoreMesh(core_axis_name="c", subcore_axis_name="s")


def _sc_degree(dst, ones16, zeros16, n, e):
    e_half = e // 2
    per_tile = e_half // 16
    nchunks = per_tile // _CHUNK
    rows_per_tile = n // 16

    @functools.partial(
        pl.kernel,
        out_type=jax.ShapeDtypeStruct((2, n, 16), jnp.float32),
        mesh=_sc_mesh(),
        scratch_types=[
            pltpu.VMEM((_CHUNK,), jnp.int32),
            pltpu.VMEM((_CHUNK, 16), jnp.float32),
            pltpu.VMEM_SHARED((n, 16), jnp.float32),
        ],
        compiler_params=pltpu.CompilerParams(use_tc_tiling_on_sc=False),
    )
    def k(dst_hbm, ones_hbm, z_hbm, out_hbm, didx, ones_v, accum):
        c = lax.axis_index("c")
        s = lax.axis_index("s")
        r0 = s * rows_per_tile
        pltpu.sync_copy(z_hbm.at[pl.ds(r0, rows_per_tile)],
                        accum.at[pl.ds(r0, rows_per_tile)])
        pltpu.sync_copy(ones_hbm, ones_v)
        plsc.subcore_barrier()
        base = c * e_half + s * per_tile

        def body(g, carry):
            pltpu.sync_copy(dst_hbm.at[pl.ds(base + g * _CHUNK, _CHUNK)], didx)
            pltpu.sync_copy(ones_v, accum.at[didx], add=True)
            return carry

        lax.fori_loop(0, nchunks, body, 0)
        plsc.subcore_barrier()
        pltpu.sync_copy(accum.at[pl.ds(r0, rows_per_tile)],
                        out_hbm.at[c, pl.ds(r0, rows_per_tile)])

    return k(dst, ones16, zeros16)


def _sc_edge_stream(tab_a, tab_b, src, dst, zeros, n, e):
    per_tile = e // 16
    nchunks = per_tile // _CHUNK
    rows_per_tile = n // 16
    w = 64

    @functools.partial(
        pl.kernel,
        out_type=jax.ShapeDtypeStruct((2, n, w), jnp.float32),
        mesh=_sc_mesh(),
        scratch_types=[
            pltpu.VMEM((_CHUNK,), jnp.int32),
            pltpu.VMEM((_CHUNK,), jnp.int32),
            pltpu.VMEM((_CHUNK, w), jnp.float32),
            pltpu.VMEM_SHARED((n, w), jnp.float32),
            pltpu.SemaphoreType.DMA,
        ],
        compiler_params=pltpu.CompilerParams(use_tc_tiling_on_sc=False),
    )
    def k(ta_hbm, tb_hbm, src_hbm, dst_hbm, z_hbm, out_hbm,
          sidx, didx, rows, accum, gsem):
        c = lax.axis_index("c")
        s = lax.axis_index("s")
        r0 = s * rows_per_tile
        pltpu.sync_copy(z_hbm.at[pl.ds(r0, rows_per_tile)],
                        accum.at[pl.ds(r0, rows_per_tile)])
        plsc.subcore_barrier()
        base = s * per_tile

        def run(tab_hbm):
            def body(g, carry):
                off = base + g * _CHUNK
                pltpu.sync_copy(src_hbm.at[pl.ds(off, _CHUNK)], sidx)
                pltpu.sync_copy(dst_hbm.at[pl.ds(off, _CHUNK)], didx)
                pltpu.async_copy(tab_hbm.at[sidx], rows, gsem).wait()
                pltpu.sync_copy(rows, accum.at[didx], add=True)
                return carry

            lax.fori_loop(0, nchunks, body, 0)

        @pl.when(c == 0)
        def _():
            run(ta_hbm)

        @pl.when(c == 1)
        def _():
            run(tb_hbm)

        plsc.subcore_barrier()
        pltpu.sync_copy(accum.at[pl.ds(r0, rows_per_tile)],
                        out_hbm.at[c, pl.ds(r0, rows_per_tile)])

    return k(tab_a, tab_b, src, dst, zeros)


_BN = 3616


def _leaky(v):
    return jnp.maximum(v, 0.2 * v)


def _dinv(dA, dB):
    return lax.rsqrt(dA[:, 0:1] + dB[:, 0:1] + 1.0)


def _tc_conv1_table(xp, sf, dA, dB, W1p, n):
    def body(xp_ref, sf_ref, da_ref, db_ref, w_ref, oa_ref, ob_ref):
        dinv = _dinv(da_ref[...], db_ref[...])
        h0 = xp_ref[...] * sf_ref[...]
        g1 = jnp.dot(h0, w_ref[...], preferred_element_type=jnp.float32)
        t1 = g1 * dinv
        oa_ref[...] = t1[:, :64]
        ob_ref[...] = t1[:, 64:]

    grid = n // _BN
    bs16 = pl.BlockSpec((_BN, 16), lambda i: (i, 0))
    bs64 = pl.BlockSpec((_BN, 64), lambda i: (i, 0))
    full = lambda r, c: pl.BlockSpec((r, c), lambda i: (0, 0))
    return pl.pallas_call(
        body,
        grid=(grid,),
        in_specs=[bs16, bs16, bs16, bs16, full(16, 128)],
        out_specs=[bs64, bs64],
        out_shape=[jax.ShapeDtypeStruct((n, 64), jnp.float32),
                   jax.ShapeDtypeStruct((n, 64), jnp.float32)],
    )(xp, sf, dA, dB, W1p)


def _tc_mid(a1A, a1B, t1a, t1b, dA, dB, b1, W2, n):
    def body(aA, aB, ta, tb, da, db, b1_ref, w2_ref, oa_ref, ob_ref):
        dinv = _dinv(da[...], db[...])
        b1v = b1_ref[...]
        h1a = _leaky((aA[...] + ta[...]) * dinv + b1v[:, :64])
        h1b = _leaky((aB[...] + tb[...]) * dinv + b1v[:, 64:])
        h1 = jnp.concatenate([h1a, h1b], axis=1)
        t2 = jnp.dot(h1, w2_ref[...], preferred_element_type=jnp.float32)
        t2 = t2 * dinv
        oa_ref[...] = t2[:, :64]
        ob_ref[...] = t2[:, 64:]

    grid = n // _BN
    bs16 = pl.BlockSpec((_BN, 16), lambda i: (i, 0))
    bs64 = pl.BlockSpec((_BN, 64), lambda i: (i, 0))
    full = lambda r, c: pl.BlockSpec((r, c), lambda i: (0, 0))
    return pl.pallas_call(
        body,
        grid=(grid,),
        in_specs=[bs64, bs64, bs64, bs64, bs16, bs16, full(1, 128),
                  full(128, 128)],
        out_specs=[bs64, bs64],
        out_shape=[jax.ShapeDtypeStruct((n, 64), jnp.float32),
                   jax.ShapeDtypeStruct((n, 64), jnp.float32)],
    )(a1A, a1B, t1a, t1b, dA, dB, b1, W2)


def _tc_tail(a2A, a2B, t2a, t2b, dA, dB, batch2, b2, gate_W, gate_b,
             p1_W, p1_b, bn_g, bn_b, p2_W, p2_b, n, num_graphs):
    grid = n // _BN
    G = num_graphs

    def body(aA, aB, ta, tb, da, db, bt, b2_ref, gw, gb,
             p1w, p1b, bg, bb, p2w, p2b, o_ref, zacc, dacc):
        i = pl.program_id(0)
        dinv = _dinv(da[...], db[...])
        b2v = b2_ref[...]
        h2a = _leaky((aA[...] + ta[...]) * dinv + b2v[:, :64])
        h2b = _leaky((aB[...] + tb[...]) * dinv + b2v[:, 64:])
        h2 = jnp.concatenate([h2a, h2b], axis=1)
        glog = jnp.dot(h2, gw[...], preferred_element_type=jnp.float32) + gb[...]
        gate = 1.0 / (1.0 + jnp.exp(-glog))
        eg = jnp.exp(gate)
        onehot = (bt[...] == lax.broadcasted_iota(jnp.int32, (_BN, G), 1))
        onehot = onehot.astype(jnp.float32)
        zc = lax.dot_general(onehot, h2 * eg, (((0,), (0,)), ((), ())),
                             preferred_element_type=jnp.float32,
                             precision=lax.Precision.HIGHEST)
        dc = lax.dot_general(onehot, eg, (((0,), (0,)), ((), ())),
                             preferred_element_type=jnp.float32,
                             precision=lax.Precision.HIGHEST)

        @pl.when(i == 0)
        def _():
            zacc[...] = jnp.zeros_like(zacc)
            dacc[...] = jnp.zeros_like(dacc)

        zacc[...] += zc
        dacc[...] += dc

        @pl.when(i == grid - 1)
        def _():
            z = zacc[...] / (dacc[...] + 1e-16)
            z1 = jnp.dot(z, p1w[...], preferred_element_type=jnp.float32) + p1b[...]
            mu = jnp.mean(z1, axis=0, keepdims=True)
            var = jnp.mean((z1 - mu) ** 2, axis=0, keepdims=True)
            z1 = (z1 - mu) / jnp.sqrt(var + 1e-5) * bg[...] + bb[...]
            z1 = _leaky(z1)
            z2 = jnp.dot(z1, p2w[...], preferred_element_type=jnp.float32) + p2b[...]
            nrm = jnp.sqrt(jnp.sum(z2 * z2, axis=1, keepdims=True))
            o_ref[...] = z2 / jnp.maximum(nrm, 1e-12)

    bs64 = pl.BlockSpec((_BN, 64), lambda i: (i, 0))
    bs16 = pl.BlockSpec((_BN, 16), lambda i: (i, 0))
    bs1 = pl.BlockSpec((_BN, 1), lambda i: (i, 0))
    full = lambda r, c: pl.BlockSpec((r, c), lambda i: (0, 0))
    return pl.pallas_call(
        body,
        grid=(grid,),
        in_specs=[bs64, bs64, bs64, bs64, bs16, bs16, bs1,
                  full(1, 128), full(128, 1), full(1, 1),
                  full(128, 512), full(1, 512), full(1, 512), full(1, 512),
                  full(512, 1024), full(1, 1024)],
        out_specs=full(G, 1024),
        out_shape=jax.ShapeDtypeStruct((G, 1024), jnp.float32),
        scratch_shapes=[pltpu.VMEM((G, 128), jnp.float32),
                        pltpu.VMEM((G, 1), jnp.float32)],
    )(a2A, a2B, t2a, t2b, dA, dB, batch2, b2, gate_W, gate_b,
      p1_W, p1_b, bn_g, bn_b, p2_W, p2_b)


def kernel(x, edge_index, batch, roi_scaler, W1, b1, W2, b2, gate_W, gate_b,
           p1_W, p1_b, bn_gamma, bn_beta, p2_W, p2_b):
    n = x.shape[0]
    e = edge_index.shape[1]
    num_graphs = n // roi_scaler.shape[0]

    src = edge_index[0]
    dst = edge_index[1]

    xp = jnp.pad(x, ((0, 0), (0, 13)))
    sf = jnp.pad(jnp.tile(roi_scaler, (num_graphs, 1)), ((0, 0), (0, 13)))
    W1p = jnp.pad(W1, ((0, 13), (0, 0)))
    zeros16 = jnp.zeros((n, 16), jnp.float32)
    zeros64 = jnp.zeros((n, 64), jnp.float32)
    ones16 = jnp.zeros((_CHUNK, 16), jnp.float32).at[:, 0].set(1.0)
    batch2 = batch.reshape(n, 1)

    deg = _sc_degree(dst, ones16, zeros16, n, e)
    dA, dB = deg[0], deg[1]

    t1a, t1b = _tc_conv1_table(xp, sf, dA, dB, W1p, n)

    acc1 = _sc_edge_stream(t1a, t1b, src, dst, zeros64, n, e)

    t2a, t2b = _tc_mid(acc1[0], acc1[1], t1a, t1b, dA, dB,
                       b1.reshape(1, 128), W2, n)

    acc2 = _sc_edge_stream(t2a, t2b, src, dst, zeros64, n, e)

    return _tc_tail(acc2[0], acc2[1], t2a, t2b, dA, dB, batch2,
                    b2.reshape(1, 128), gate_W, gate_b.reshape(1, 1),
                    p1_W, p1_b.reshape(1, 512), bn_gamma.reshape(1, 512),
                    bn_beta.reshape(1, 512), p2_W, p2_b.reshape(1, 1024),
                    n, num_graphs)


# speedup vs baseline: 16.5545x; 1.8963x over previous
import functools

import jax
import jax.numpy as jnp
from jax import lax
from jax.experimental import pallas as pl
from jax.experimental.pallas import tpu as pltpu
from jax.experimental.pallas import tpu_sc as plsc

_CK = 113
_GC = 16
_NR = 8
_LG = 4
_W = 32


def _sc_mesh():
    return plsc.VectorSubcoreMesh(core_axis_name="c", subcore_axis_name="s")


def _sc_degree(dst2, ones16, zeros16, n, e):
    nrows = e // _CK
    rows_per_tile = nrows // 32
    ngroups = rows_per_tile // _NR
    out_rows = n // 16

    @functools.partial(
        pl.kernel,
        out_type=[jax.ShapeDtypeStruct((n, 16), jnp.float32),
                  jax.ShapeDtypeStruct((n, 16), jnp.float32)],
        mesh=_sc_mesh(),
        scratch_types=[
            pltpu.VMEM((rows_per_tile, _CK), jnp.int32),
            pltpu.VMEM((_CK, 16), jnp.float32),
            pltpu.VMEM_SHARED((n, 16), jnp.float32),
            pltpu.SemaphoreType.DMA((_NR,)),
        ],
        compiler_params=pltpu.CompilerParams(use_tc_tiling_on_sc=False),
    )
    def k(dst_hbm, ones_hbm, z_hbm, outa, outb, didx, ones_v, accum, ssem):
        c = lax.axis_index("c")
        s = lax.axis_index("s")
        r0 = s * out_rows
        pltpu.sync_copy(z_hbm.at[pl.ds(r0, out_rows)],
                        accum.at[pl.ds(r0, out_rows)])
        pltpu.sync_copy(ones_hbm, ones_v)
        base = c * (nrows // 2) + s * rows_per_tile
        pltpu.sync_copy(dst_hbm.at[pl.ds(base, rows_per_tile)], didx)
        plsc.subcore_barrier()

        def fire(g, b):
            pltpu.async_copy(ones_v, accum.at[didx.at[g]], ssem.at[b],
                             add=True)

        def drain(b):
            pltpu.make_async_copy(z_hbm.at[pl.ds(0, _CK)], ones_v,
                                  ssem.at[b]).wait()

        for b in range(_NR):
            fire(b, b)

        def group(gg, carry):
            for b in range(_NR):
                drain(b)
                fire(gg * _NR + b, b)
            return carry

        lax.fori_loop(1, ngroups, group, 0)
        for b in range(_NR):
            drain(b)
        plsc.subcore_barrier()

        @pl.when(c == 0)
        def _():
            pltpu.sync_copy(accum.at[pl.ds(r0, out_rows)],
                            outa.at[pl.ds(r0, out_rows)])

        @pl.when(c == 1)
        def _():
            pltpu.sync_copy(accum.at[pl.ds(r0, out_rows)],
                            outb.at[pl.ds(r0, out_rows)])

    return k(dst2, ones16, zeros16)


def _sc_edge_stream(t0, t1, t2, t3, src2, dst2, n, e):
    nrows = e // _CK
    rows_per_tile = nrows // 16
    ngroups = rows_per_tile // _GC
    out_rows = n // 16

    @functools.partial(
        pl.kernel,
        out_type=[jax.ShapeDtypeStruct((n, _W), jnp.float32)] * 4,
        mesh=_sc_mesh(),
        scratch_types=[
            pltpu.VMEM((2, _GC, _CK), jnp.int32),
            pltpu.VMEM((2, _GC, _CK), jnp.int32),
            pltpu.VMEM((_NR, _CK, _W), jnp.float32),
            pltpu.VMEM_SHARED((n, _W), jnp.float32),
            pltpu.SemaphoreType.DMA((2,)),
            pltpu.SemaphoreType.DMA((_NR,)),
            pltpu.SemaphoreType.DMA((_NR,)),
        ],
        compiler_params=pltpu.CompilerParams(use_tc_tiling_on_sc=False),
    )
    def k(t0h, t1h, t2h, t3h, src_hbm, dst_hbm, o0, o1, o2, o3,
          sidx, didx, rows, accum, isem, gsem, ssem):
        c = lax.axis_index("c")
        s = lax.axis_index("s")
        r0 = s * out_rows
        base = s * rows_per_tile

        def do_pass(tab, out):
            def fire_i(G, p):
                pltpu.async_copy(src_hbm.at[pl.ds(base + G * _GC, _GC)],
                                 sidx.at[p], isem.at[p])
                pltpu.async_copy(dst_hbm.at[pl.ds(base + G * _GC, _GC)],
                                 didx.at[p], isem.at[p])

            def wait_i(p):
                pltpu.make_async_copy(src_hbm.at[pl.ds(0, _GC)],
                                      sidx.at[p], isem.at[p]).wait()
                pltpu.make_async_copy(dst_hbm.at[pl.ds(0, _GC)],
                                      didx.at[p], isem.at[p]).wait()

            def fire_g(p, j, b):
                pltpu.async_copy(tab.at[sidx.at[p, j]], rows.at[b],
                                 gsem.at[b])

            def wait_g(b):
                pltpu.make_async_copy(tab.at[pl.ds(0, _CK)], rows.at[b],
                                      gsem.at[b]).wait()

            def fire_s(p, j, b):
                pltpu.async_copy(rows.at[b], accum.at[didx.at[p, j]],
                                 ssem.at[b], add=True)

            def wait_s(b):
                pltpu.make_async_copy(tab.at[pl.ds(0, _CK)], rows.at[b],
                                      ssem.at[b]).wait()

            pltpu.sync_copy(tab.at[pl.ds(r0, out_rows)],
                            accum.at[pl.ds(r0, out_rows)])
            pltpu.sync_copy(src_hbm.at[pl.ds(base, _GC)], sidx.at[0])
            pltpu.sync_copy(dst_hbm.at[pl.ds(base, _GC)], didx.at[0])
            for j in range(_LG):
                fire_g(0, j, j)
            plsc.subcore_barrier()

            def step(G, p, pn, j, first, last):
                b = j % _NR
                if not (first and j < _LG):
                    wait_s((j - _LG) % _NR)
                wait_g(b)
                fire_s(p, j, b)
                if j == 4 and not last:
                    fire_i(G + 1, pn)
                if j == 12 and not last:
                    wait_i(pn)
                if j < _GC - _LG:
                    fire_g(p, j + _LG, (j + _LG) % _NR)
                elif not last:
                    fire_g(pn, j - (_GC - _LG), (j + _LG) % _NR)

            for j in range(_GC):
                step(0, 0, 1, j, True, False)

            def group(G, carry):
                p = lax.rem(G, 2)
                pn = 1 - p
                for j in range(_GC):
                    step(G, p, pn, j, False, False)
                return carry

            lax.fori_loop(1, ngroups - 1, group, 0)

            pl_ = (ngroups - 1) % 2
            for j in range(_GC):
                step(ngroups - 1, pl_, 1 - pl_, j, False, True)
            for b in range(_GC - _LG, _GC):
                wait_s(b % _NR)

            plsc.subcore_barrier()
            pltpu.sync_copy(accum.at[pl.ds(r0, out_rows)],
                            out.at[pl.ds(r0, out_rows)])

        @pl.when(c == 0)
        def _():
            do_pass(t0h, o0)
            do_pass(t1h, o1)

        @pl.when(c == 1)
        def _():
            do_pass(t2h, o2)
            do_pass(t3h, o3)

    return k(t0, t1, t2, t3, src2, dst2)


_BN = 3616


def _leaky(v):
    return jnp.maximum(v, 0.2 * v)


def _dinv(dA, dB):
    return lax.rsqrt(dA[:, 0:1] + dB[:, 0:1] + 1.0)


def _quarters(t):
    return t[:, 0:32], t[:, 32:64], t[:, 64:96], t[:, 96:128]


def _tc_conv1_table(xp, sf, dA, dB, W1p, n):
    def body(xp_ref, sf_ref, da_ref, db_ref, w_ref, o0, o1, o2, o3):
        dinv = _dinv(da_ref[...], db_ref[...])
        h0 = xp_ref[...] * sf_ref[...]
        g1 = jnp.dot(h0, w_ref[...], preferred_element_type=jnp.float32)
        t1 = g1 * dinv
        q = _quarters(t1)
        o0[...], o1[...], o2[...], o3[...] = q

    grid = n // _BN
    bs16 = pl.BlockSpec((_BN, 16), lambda i: (i, 0))
    bs32 = pl.BlockSpec((_BN, _W), lambda i: (i, 0))
    full = lambda r, c: pl.BlockSpec((r, c), lambda i: (0, 0))
    return pl.pallas_call(
        body,
        grid=(grid,),
        in_specs=[bs16, bs16, bs16, bs16, full(16, 128)],
        out_specs=[bs32] * 4,
        out_shape=[jax.ShapeDtypeStruct((n, _W), jnp.float32)] * 4,
    )(xp, sf, dA, dB, W1p)


def _tc_mid(a0, a1, a2, a3, dA, dB, b1, W2, n):
    def body(a0r, a1r, a2r, a3r, da, db, b1_ref, w2_ref, o0, o1, o2, o3):
        dinv = _dinv(da[...], db[...])
        b1v = b1_ref[...]
        h1 = jnp.concatenate(
            [_leaky(a0r[...] * dinv + b1v[:, 0:32]),
             _leaky(a1r[...] * dinv + b1v[:, 32:64]),
             _leaky(a2r[...] * dinv + b1v[:, 64:96]),
             _leaky(a3r[...] * dinv + b1v[:, 96:128])], axis=1)
        t2 = jnp.dot(h1, w2_ref[...], preferred_element_type=jnp.float32)
        t2 = t2 * dinv
        q = _quarters(t2)
        o0[...], o1[...], o2[...], o3[...] = q

    grid = n // _BN
    bs16 = pl.BlockSpec((_BN, 16), lambda i: (i, 0))
    bs32 = pl.BlockSpec((_BN, _W), lambda i: (i, 0))
    full = lambda r, c: pl.BlockSpec((r, c), lambda i: (0, 0))
    return pl.pallas_call(
        body,
        grid=(grid,),
        in_specs=[bs32, bs32, bs32, bs32, bs16, bs16, full(1, 128),
                  full(128, 128)],
        out_specs=[bs32] * 4,
        out_shape=[jax.ShapeDtypeStruct((n, _W), jnp.float32)] * 4,
    )(a0, a1, a2, a3, dA, dB, b1, W2)


def _tc_tail(a0, a1, a2, a3, dA, dB, batch2, b2, gate_W, gate_b,
             p1_W, p1_b, bn_g, bn_b, p2_W, p2_b, n, num_graphs):
    grid = n // _BN
    G = num_graphs

    def body(a0r, a1r, a2r, a3r, da, db, bt, b2_ref, gw, gb,
             p1w, p1b, bg, bb, p2w, p2b, o_ref, zacc, dacc):
        i = pl.program_id(0)
        dinv = _dinv(da[...], db[...])
        b2v = b2_ref[...]
        h2 = jnp.concatenate(
            [_leaky(a0r[...] * dinv + b2v[:, 0:32]),
             _leaky(a1r[...] * dinv + b2v[:, 32:64]),
             _leaky(a2r[...] * dinv + b2v[:, 64:96]),
             _leaky(a3r[...] * dinv + b2v[:, 96:128])], axis=1)
        glog = jnp.dot(h2, gw[...], preferred_element_type=jnp.float32) + gb[...]
        gate = 1.0 / (1.0 + jnp.exp(-glog))
        eg = jnp.exp(gate)
        onehot = (bt[...] == lax.broadcasted_iota(jnp.int32, (_BN, G), 1))
        onehot = onehot.astype(jnp.float32)
        zc = lax.dot_general(onehot, h2 * eg, (((0,), (0,)), ((), ())),
                             preferred_element_type=jnp.float32,
                             precision=lax.Precision.HIGHEST)
        dc = lax.dot_general(onehot, eg, (((0,), (0,)), ((), ())),
                             preferred_element_type=jnp.float32,
                             precision=lax.Precision.HIGHEST)

        @pl.when(i == 0)
        def _():
            zacc[...] = jnp.zeros_like(zacc)
            dacc[...] = jnp.zeros_like(dacc)

        zacc[...] += zc
        dacc[...] += dc

        @pl.when(i == grid - 1)
        def _():
            z = zacc[...] / (dacc[...] + 1e-16)
            z1 = jnp.dot(z, p1w[...], preferred_element_type=jnp.float32) + p1b[...]
            mu = jnp.mean(z1, axis=0, keepdims=True)
            var = jnp.mean((z1 - mu) ** 2, axis=0, keepdims=True)
            z1 = (z1 - mu) / jnp.sqrt(var + 1e-5) * bg[...] + bb[...]
            z1 = _leaky(z1)
            z2 = jnp.dot(z1, p2w[...], preferred_element_type=jnp.float32) + p2b[...]
            nrm = jnp.sqrt(jnp.sum(z2 * z2, axis=1, keepdims=True))
            o_ref[...] = z2 / jnp.maximum(nrm, 1e-12)

    bs32 = pl.BlockSpec((_BN, _W), lambda i: (i, 0))
    bs16 = pl.BlockSpec((_BN, 16), lambda i: (i, 0))
    bs1 = pl.BlockSpec((_BN, 1), lambda i: (i, 0))
    full = lambda r, c: pl.BlockSpec((r, c), lambda i: (0, 0))
    return pl.pallas_call(
        body,
        grid=(grid,),
        in_specs=[bs32, bs32, bs32, bs32, bs16, bs16, bs1,
                  full(1, 128), full(128, 1), full(1, 1),
                  full(128, 512), full(1, 512), full(1, 512), full(1, 512),
                  full(512, 1024), full(1, 1024)],
        out_specs=full(G, 1024),
        out_shape=jax.ShapeDtypeStruct((G, 1024), jnp.float32),
        scratch_shapes=[pltpu.VMEM((G, 128), jnp.float32),
                        pltpu.VMEM((G, 1), jnp.float32)],
    )(a0, a1, a2, a3, dA, dB, batch2, b2, gate_W, gate_b,
      p1_W, p1_b, bn_g, bn_b, p2_W, p2_b)


def kernel(x, edge_index, batch, roi_scaler, W1, b1, W2, b2, gate_W, gate_b,
           p1_W, p1_b, bn_gamma, bn_beta, p2_W, p2_b):
    n = x.shape[0]
    e = edge_index.shape[1]
    num_graphs = n // roi_scaler.shape[0]

    src2 = edge_index[0].reshape(e // _CK, _CK)
    dst2 = edge_index[1].reshape(e // _CK, _CK)
    xp = jnp.pad(x, ((0, 0), (0, 13)))
    sf = jnp.pad(jnp.tile(roi_scaler, (num_graphs, 1)), ((0, 0), (0, 13)))
    W1p = jnp.pad(W1, ((0, 13), (0, 0)))
    zeros16 = jnp.zeros((n, 16), jnp.float32)
    ones16 = jnp.zeros((_CK, 16), jnp.float32).at[:, 0].set(1.0)
    batch2 = batch.reshape(n, 1)

    dA, dB = _sc_degree(dst2, ones16, zeros16, n, e)

    t1 = _tc_conv1_table(xp, sf, dA, dB, W1p, n)

    a1 = _sc_edge_stream(*t1, src2, dst2, n, e)

    t2 = _tc_mid(*a1, dA, dB, b1.reshape(1, 128), W2, n)

    a2 = _sc_edge_stream(*t2, src2, dst2, n, e)

    return _tc_tail(*a2, dA, dB, batch2,
                    b2.reshape(1, 128), gate_W, gate_b.reshape(1, 1),
                    p1_W, p1_b.reshape(1, 512), bn_gamma.reshape(1, 512),
                    bn_beta.reshape(1, 512), p2_W, p2_b.reshape(1, 1024),
                    n, num_graphs)


# speedup vs baseline: 22.7821x; 1.3762x over previous
import functools

import jax
import jax.numpy as jnp
from jax import lax
from jax.experimental import pallas as pl
from jax.experimental.pallas import tpu as pltpu
from jax.experimental.pallas import tpu_sc as plsc

_CK = 128
_GC = 16
_NR = 8
_LG = 4
_W = 32


def _sc_mesh():
    return plsc.VectorSubcoreMesh(core_axis_name="c", subcore_axis_name="s")


def _sc_degree(dst2, ones1, zeros1, n, e):
    nrows = e // _CK
    rows_per_tile = nrows // 32
    out_rows = n // 16

    @functools.partial(
        pl.kernel,
        out_type=[jax.ShapeDtypeStruct((n,), jnp.float32),
                  jax.ShapeDtypeStruct((n,), jnp.float32)],
        mesh=_sc_mesh(),
        scratch_types=[
            pltpu.VMEM((rows_per_tile, _CK), jnp.int32),
            pltpu.VMEM((_CK,), jnp.float32),
            pltpu.VMEM_SHARED((n,), jnp.float32),
            pltpu.SemaphoreType.DMA((_NR,)),
        ],
        compiler_params=pltpu.CompilerParams(use_tc_tiling_on_sc=False),
    )
    def k(dst_hbm, ones_hbm, z_hbm, outa, outb, didx, ones_v, accum, ssem):
        c = lax.axis_index("c")
        s = lax.axis_index("s")
        r0 = s * out_rows
        pltpu.sync_copy(z_hbm.at[pl.ds(r0, out_rows)],
                        accum.at[pl.ds(r0, out_rows)])
        pltpu.sync_copy(ones_hbm, ones_v)
        base = c * (nrows // 2) + s * rows_per_tile
        pltpu.sync_copy(dst_hbm.at[pl.ds(base, rows_per_tile)], didx)
        plsc.subcore_barrier()

        def fire(g, b):
            pltpu.async_copy(ones_v, accum.at[didx.at[g]], ssem.at[b],
                             add=True)

        def drain(b):
            pltpu.make_async_copy(z_hbm.at[pl.ds(0, _CK)], ones_v,
                                  ssem.at[b]).wait()

        for b in range(_NR):
            fire(b, b)

        def group(gg, carry):
            for b in range(_NR):
                drain(b)
                fire(gg * _NR + b, b)
            return carry

        lax.fori_loop(1, 14, group, 0)
        drain(0)
        fire(112, 0)
        for b in range(_NR):
            drain(b)
        plsc.subcore_barrier()

        @pl.when(c == 0)
        def _():
            pltpu.sync_copy(accum.at[pl.ds(r0, out_rows)],
                            outa.at[pl.ds(r0, out_rows)])

        @pl.when(c == 1)
        def _():
            pltpu.sync_copy(accum.at[pl.ds(r0, out_rows)],
                            outb.at[pl.ds(r0, out_rows)])

    return k(dst2, ones1, zeros1)


def _sc_edge_stream(tab4, zeros32, src2, dst2, n, e):
    nrows = e // _CK
    rows_per_tile = nrows // 16
    nfull = 14
    out_rows = n // 16

    @functools.partial(
        pl.kernel,
        out_type=jax.ShapeDtypeStruct((n, 128), jnp.float32),
        mesh=_sc_mesh(),
        scratch_types=[
            pltpu.VMEM((2, _GC, _CK), jnp.int32),
            pltpu.VMEM((2, _GC, _CK), jnp.int32),
            pltpu.VMEM((2, _GC, _CK), jnp.int32),
            pltpu.VMEM((_NR, _CK, _W), jnp.float32),
            pltpu.VMEM_SHARED((n, _W), jnp.float32),
            pltpu.SemaphoreType.DMA((2,)),
            pltpu.SemaphoreType.DMA((_NR,)),
            pltpu.SemaphoreType.DMA((_NR,)),
        ],
        compiler_params=pltpu.CompilerParams(use_tc_tiling_on_sc=False),
    )
    def k(tab4_hbm, z_hbm, src_hbm, dst_hbm, out_hbm,
          sidx, sidx4, didx, rows, accum, isem, gsem, ssem):
        c = lax.axis_index("c")
        s = lax.axis_index("s")
        r0 = s * out_rows
        base = s * rows_per_tile

        def do_pass(q):
            def fire_i(G, p):
                pltpu.async_copy(src_hbm.at[pl.ds(base + G * _GC, _GC)],
                                 sidx.at[p], isem.at[p])
                pltpu.async_copy(dst_hbm.at[pl.ds(base + G * _GC, _GC)],
                                 didx.at[p], isem.at[p])

            def wait_i(p):
                pltpu.make_async_copy(src_hbm.at[pl.ds(0, _GC)],
                                      sidx.at[p], isem.at[p]).wait()
                pltpu.make_async_copy(dst_hbm.at[pl.ds(0, _GC)],
                                      didx.at[p], isem.at[p]).wait()

            def xform(p):
                def tbody(cc, carry):
                    for kk in range(_CK // 16):
                        v = sidx[p, cc, pl.ds(kk * 16, 16)]
                        sidx4[p, cc, pl.ds(kk * 16, 16)] = v * 4 + q
                    return carry

                lax.fori_loop(0, _GC, tbody, 0)

            def fire_g(p, j, b):
                pltpu.async_copy(tab4_hbm.at[sidx4.at[p, j]], rows.at[b],
                                 gsem.at[b])

            def wait_g(b):
                pltpu.make_async_copy(tab4_hbm.at[pl.ds(0, _CK)], rows.at[b],
                                      gsem.at[b]).wait()

            def fire_s(p, j, b):
                pltpu.async_copy(rows.at[b], accum.at[didx.at[p, j]],
                                 ssem.at[b], add=True)

            def wait_s(b):
                pltpu.make_async_copy(tab4_hbm.at[pl.ds(0, _CK)], rows.at[b],
                                      ssem.at[b]).wait()

            pltpu.sync_copy(z_hbm.at[pl.ds(r0, out_rows)],
                            accum.at[pl.ds(r0, out_rows)])
            pltpu.sync_copy(src_hbm.at[pl.ds(base, _GC)], sidx.at[0])
            pltpu.sync_copy(dst_hbm.at[pl.ds(base, _GC)], didx.at[0])
            xform(0)
            for j in range(_LG):
                fire_g(0, j, j)
            plsc.subcore_barrier()

            def step(G, p, pn, j, first, last):
                b = j % _NR
                if not (first and j < _LG):
                    wait_s((j - _LG) % _NR)
                wait_g(b)
                fire_s(p, j, b)
                if j == 4:
                    fire_i(G + 1, pn)
                if j == 12:
                    wait_i(pn)
                    xform(pn)
                if j < _GC - _LG:
                    fire_g(p, j + _LG, (j + _LG) % _NR)
                elif not last:
                    fire_g(pn, j - (_GC - _LG), (j + _LG) % _NR)
                elif j - (_GC - _LG) < 2:
                    fire_g(pn, j - (_GC - _LG), (j + _LG) % _NR)

            for j in range(_GC):
                step(0, 0, 1, j, True, False)

            def group(G, carry):
                p = lax.rem(G, 2)
                pn = 1 - p
                for j in range(_GC):
                    step(G, p, pn, j, False, False)
                return carry

            lax.fori_loop(1, nfull - 1, group, 0)

            pl_ = (nfull - 1) % 2
            for j in range(_GC):
                step(nfull - 1, pl_, 1 - pl_, j, False, True)

            pt = nfull % 2
            for t in range(2):
                g = nfull * _GC + t
                b = g % _NR
                wait_s((b - _LG) % _NR)
                wait_g(b)
                fire_s(pt, t, b)
            for b in (6, 7, 0, 1):
                wait_s(b)

            plsc.subcore_barrier()
            pltpu.sync_copy(accum.at[pl.ds(r0, out_rows)],
                            out_hbm.at[pl.ds(r0, out_rows),
                                       pl.ds(q * _W, _W)])

        @pl.when(c == 0)
        def _():
            do_pass(0)
            do_pass(1)

        @pl.when(c == 1)
        def _():
            do_pass(2)
            do_pass(3)

    return k(tab4, zeros32, src2, dst2)


_BN = 3616


def _leaky(v):
    return jnp.maximum(v, 0.2 * v)


def _dinv(dA, dB):
    return lax.rsqrt(dA + dB + 1.0)


def _tc_conv1_table(x, sf, dA, dB, W1, n):
    def body(x_ref, sf_ref, da_ref, db_ref, w_ref, o_ref):
        dinv = _dinv(da_ref[...], db_ref[...])
        h0 = x_ref[...] * sf_ref[...]
        g1 = jnp.dot(h0, w_ref[...], preferred_element_type=jnp.float32)
        o_ref[...] = g1 * dinv

    grid = n // _BN
    bs3 = pl.BlockSpec((_BN, 3), lambda i: (i, 0))
    bs1 = pl.BlockSpec((_BN, 1), lambda i: (i, 0))
    bs128 = pl.BlockSpec((_BN, 128), lambda i: (i, 0))
    full = lambda r, c: pl.BlockSpec((r, c), lambda i: (0, 0))
    return pl.pallas_call(
        body,
        grid=(grid,),
        in_specs=[bs3, bs3, bs1, bs1, full(3, 128)],
        out_specs=bs128,
        out_shape=jax.ShapeDtypeStruct((n, 128), jnp.float32),
    )(x, sf, dA, dB, W1)


def _tc_mid(a1, t1, dA, dB, b1, W2, n):
    def body(a_ref, t_ref, da, db, b1_ref, w2_ref, o_ref):
        dinv = _dinv(da[...], db[...])
        h1 = _leaky((a_ref[...] + t_ref[...]) * dinv + b1_ref[...])
        t2 = jnp.dot(h1, w2_ref[...], preferred_element_type=jnp.float32)
        o_ref[...] = t2 * dinv

    grid = n // _BN
    bs1 = pl.BlockSpec((_BN, 1), lambda i: (i, 0))
    bs128 = pl.BlockSpec((_BN, 128), lambda i: (i, 0))
    full = lambda r, c: pl.BlockSpec((r, c), lambda i: (0, 0))
    return pl.pallas_call(
        body,
        grid=(grid,),
        in_specs=[bs128, bs128, bs1, bs1, full(1, 128), full(128, 128)],
        out_specs=bs128,
        out_shape=jax.ShapeDtypeStruct((n, 128), jnp.float32),
    )(a1, t1, dA, dB, b1, W2)


def _tc_tail(a2, t2, dA, dB, batch2, b2, gate_W, gate_b,
             p1_W, p1_b, bn_g, bn_b, p2_W, p2_b, n, num_graphs):
    grid = n // _BN
    G = num_graphs

    def body(a_ref, t_ref, da, db, bt, b2_ref, gw, gb,
             p1w, p1b, bg, bb, p2w, p2b, o_ref, zacc, dacc):
        i = pl.program_id(0)
        dinv = _dinv(da[...], db[...])
        h2 = _leaky((a_ref[...] + t_ref[...]) * dinv + b2_ref[...])
        glog = jnp.dot(h2, gw[...], preferred_element_type=jnp.float32) + gb[...]
        gate = 1.0 / (1.0 + jnp.exp(-glog))
        eg = jnp.exp(gate)
        onehot = (bt[...] == lax.broadcasted_iota(jnp.int32, (_BN, G), 1))
        onehot = onehot.astype(jnp.float32)
        zc = lax.dot_general(onehot, h2 * eg, (((0,), (0,)), ((), ())),
                             preferred_element_type=jnp.float32,
                             precision=lax.Precision.HIGHEST)
        dc = lax.dot_general(onehot, eg, (((0,), (0,)), ((), ())),
                             preferred_element_type=jnp.float32,
                             precision=lax.Precision.HIGHEST)

        @pl.when(i == 0)
        def _():
            zacc[...] = jnp.zeros_like(zacc)
            dacc[...] = jnp.zeros_like(dacc)

        zacc[...] += zc
        dacc[...] += dc

        @pl.when(i == grid - 1)
        def _():
            z = zacc[...] / (dacc[...] + 1e-16)
            z1 = jnp.dot(z, p1w[...], preferred_element_type=jnp.float32) + p1b[...]
            mu = jnp.mean(z1, axis=0, keepdims=True)
            var = jnp.mean((z1 - mu) ** 2, axis=0, keepdims=True)
            z1 = (z1 - mu) / jnp.sqrt(var + 1e-5) * bg[...] + bb[...]
            z1 = _leaky(z1)
            z2 = jnp.dot(z1, p2w[...], preferred_element_type=jnp.float32) + p2b[...]
            nrm = jnp.sqrt(jnp.sum(z2 * z2, axis=1, keepdims=True))
            o_ref[...] = z2 / jnp.maximum(nrm, 1e-12)

    bs128 = pl.BlockSpec((_BN, 128), lambda i: (i, 0))
    bs1 = pl.BlockSpec((_BN, 1), lambda i: (i, 0))
    full = lambda r, c: pl.BlockSpec((r, c), lambda i: (0, 0))
    return pl.pallas_call(
        body,
        grid=(grid,),
        in_specs=[bs128, bs128, bs1, bs1, bs1,
                  full(1, 128), full(128, 1), full(1, 1),
                  full(128, 512), full(1, 512), full(1, 512), full(1, 512),
                  full(512, 1024), full(1, 1024)],
        out_specs=full(G, 1024),
        out_shape=jax.ShapeDtypeStruct((G, 1024), jnp.float32),
        scratch_shapes=[pltpu.VMEM((G, 128), jnp.float32),
                        pltpu.VMEM((G, 1), jnp.float32)],
    )(a2, t2, dA, dB, batch2, b2, gate_W, gate_b,
      p1_W, p1_b, bn_g, bn_b, p2_W, p2_b)


def kernel(x, edge_index, batch, roi_scaler, W1, b1, W2, b2, gate_W, gate_b,
           p1_W, p1_b, bn_gamma, bn_beta, p2_W, p2_b):
    n = x.shape[0]
    e = edge_index.shape[1]
    num_graphs = n // roi_scaler.shape[0]

    src2 = jnp.pad(edge_index[0].reshape(e // _CK, _CK), ((0, _GC), (0, 0)))
    dst2 = jnp.pad(edge_index[1].reshape(e // _CK, _CK), ((0, _GC), (0, 0)))
    sf = jnp.tile(roi_scaler, (num_graphs, 1))
    ones1 = jnp.ones((_CK,), jnp.float32)
    zeros1 = jnp.zeros((n,), jnp.float32)
    batch2 = batch.reshape(n, 1)
    zeros32 = jnp.zeros((n, _W), jnp.float32)

    degA, degB = _sc_degree(dst2, ones1, zeros1, n, e)
    dA = degA.reshape(n, 1)
    dB = degB.reshape(n, 1)

    t1 = _tc_conv1_table(x, sf, dA, dB, W1, n)

    a1 = _sc_edge_stream(t1.reshape(4 * n, _W), zeros32, src2, dst2, n, e)

    t2 = _tc_mid(a1, t1, dA, dB, b1.reshape(1, 128), W2, n)

    a2 = _sc_edge_stream(t2.reshape(4 * n, _W), zeros32, src2, dst2, n, e)

    return _tc_tail(a2, t2, dA, dB, batch2,
                    b2.reshape(1, 128), gate_W, gate_b.reshape(1, 1),
                    p1_W, p1_b.reshape(1, 512), bn_gamma.reshape(1, 512),
                    bn_beta.reshape(1, 512), p2_W, p2_b.reshape(1, 1024),
                    n, num_graphs)


# speedup vs baseline: 24.1425x; 1.0597x over previous
import functools

import jax
import jax.numpy as jnp
from jax import lax
from jax.experimental import pallas as pl
from jax.experimental.pallas import tpu as pltpu
from jax.experimental.pallas import tpu_sc as plsc

_CK = 128
_GC = 16
_NR = 8
_LG = 5
_WS = _NR - _LG
_W = 32


def _sc_mesh():
    return plsc.VectorSubcoreMesh(core_axis_name="c", subcore_axis_name="s")


def _sc_degree(dst2, ones1, zeros1, n, e):
    nrows = e // _CK
    rows_per_tile = nrows // 32
    out_rows = n // 16

    @functools.partial(
        pl.kernel,
        out_type=[jax.ShapeDtypeStruct((n,), jnp.float32),
                  jax.ShapeDtypeStruct((n,), jnp.float32)],
        mesh=_sc_mesh(),
        scratch_types=[
            pltpu.VMEM((rows_per_tile, _CK), jnp.int32),
            pltpu.VMEM((_CK,), jnp.float32),
            pltpu.VMEM_SHARED((n,), jnp.float32),
            pltpu.SemaphoreType.DMA((_NR,)),
        ],
        compiler_params=pltpu.CompilerParams(use_tc_tiling_on_sc=False),
    )
    def k(dst_hbm, ones_hbm, z_hbm, outa, outb, didx, ones_v, accum, ssem):
        c = lax.axis_index("c")
        s = lax.axis_index("s")
        r0 = s * out_rows
        pltpu.sync_copy(z_hbm.at[pl.ds(r0, out_rows)],
                        accum.at[pl.ds(r0, out_rows)])
        pltpu.sync_copy(ones_hbm, ones_v)
        base = c * (nrows // 2) + s * rows_per_tile
        pltpu.sync_copy(dst_hbm.at[pl.ds(base, rows_per_tile)], didx)
        plsc.subcore_barrier()

        def fire(g, b):
            pltpu.async_copy(ones_v, accum.at[didx.at[g]], ssem.at[b],
                             add=True)

        def drain(b):
            pltpu.make_async_copy(z_hbm.at[pl.ds(0, _CK)], ones_v,
                                  ssem.at[b]).wait()

        for b in range(_NR):
            fire(b, b)

        def group(gg, carry):
            for b in range(_NR):
                drain(b)
                fire(gg * _NR + b, b)
            return carry

        lax.fori_loop(1, 14, group, 0)
        drain(0)
        fire(112, 0)
        for b in range(_NR):
            drain(b)
        plsc.subcore_barrier()

        @pl.when(c == 0)
        def _():
            pltpu.sync_copy(accum.at[pl.ds(r0, out_rows)],
                            outa.at[pl.ds(r0, out_rows)])

        @pl.when(c == 1)
        def _():
            pltpu.sync_copy(accum.at[pl.ds(r0, out_rows)],
                            outb.at[pl.ds(r0, out_rows)])

    return k(dst2, ones1, zeros1)


def _sc_edge_stream(tab4, zeros32, src2, dst2, n, e):
    nrows = e // _CK
    rows_per_tile = nrows // 16
    nfull = 14
    out_rows = n // 16

    @functools.partial(
        pl.kernel,
        out_type=jax.ShapeDtypeStruct((n, 128), jnp.float32),
        mesh=_sc_mesh(),
        scratch_types=[
            pltpu.VMEM((2, _GC, _CK), jnp.int32),
            pltpu.VMEM((2, _GC, _CK), jnp.int32),
            pltpu.VMEM((2, _GC, _CK), jnp.int32),
            pltpu.VMEM((_NR, _CK, _W), jnp.float32),
            pltpu.VMEM_SHARED((n, _W), jnp.float32),
            pltpu.SemaphoreType.DMA((2,)),
            pltpu.SemaphoreType.DMA((_NR,)),
            pltpu.SemaphoreType.DMA((_NR,)),
        ],
        compiler_params=pltpu.CompilerParams(use_tc_tiling_on_sc=False),
    )
    def k(tab4_hbm, z_hbm, src_hbm, dst_hbm, out_hbm,
          sidx, sidx4, didx, rows, accum, isem, gsem, ssem):
        c = lax.axis_index("c")
        s = lax.axis_index("s")
        r0 = s * out_rows
        base = s * rows_per_tile

        def do_pass(q):
            def fire_i(G, p):
                pltpu.async_copy(src_hbm.at[pl.ds(base + G * _GC, _GC)],
                                 sidx.at[p], isem.at[p])
                pltpu.async_copy(dst_hbm.at[pl.ds(base + G * _GC, _GC)],
                                 didx.at[p], isem.at[p])

            def wait_i(p):
                pltpu.make_async_copy(src_hbm.at[pl.ds(0, _GC)],
                                      sidx.at[p], isem.at[p]).wait()
                pltpu.make_async_copy(dst_hbm.at[pl.ds(0, _GC)],
                                      didx.at[p], isem.at[p]).wait()

            def xform(p):
                def tbody(cc, carry):
                    for kk in range(_CK // 16):
                        v = sidx[p, cc, pl.ds(kk * 16, 16)]
                        sidx4[p, cc, pl.ds(kk * 16, 16)] = v * 4 + q
                    return carry

                lax.fori_loop(0, _GC, tbody, 0)

            def fire_g(p, j, b):
                pltpu.async_copy(tab4_hbm.at[sidx4.at[p, j]], rows.at[b],
                                 gsem.at[b])

            def wait_g(b):
                pltpu.make_async_copy(tab4_hbm.at[pl.ds(0, _CK)], rows.at[b],
                                      gsem.at[b]).wait()

            def fire_s(p, j, b):
                pltpu.async_copy(rows.at[b], accum.at[didx.at[p, j]],
                                 ssem.at[b], add=True)

            def wait_s(b):
                pltpu.make_async_copy(tab4_hbm.at[pl.ds(0, _CK)], rows.at[b],
                                      ssem.at[b]).wait()

            pltpu.sync_copy(z_hbm.at[pl.ds(r0, out_rows)],
                            accum.at[pl.ds(r0, out_rows)])
            pltpu.sync_copy(src_hbm.at[pl.ds(base, _GC)], sidx.at[0])
            pltpu.sync_copy(dst_hbm.at[pl.ds(base, _GC)], didx.at[0])
            xform(0)
            for j in range(_LG):
                fire_g(0, j, j)
            plsc.subcore_barrier()

            def step(G, p, pn, j, first, last):
                b = j % _NR
                if not (first and j < _WS):
                    wait_s((j - _WS) % _NR)
                wait_g(b)
                fire_s(p, j, b)
                if j == 4:
                    fire_i(G + 1, pn)
                if j == _GC - _LG - 1:
                    wait_i(pn)
                    xform(pn)
                if j < _GC - _LG:
                    fire_g(p, j + _LG, (j + _LG) % _NR)
                elif not last:
                    fire_g(pn, j - (_GC - _LG), (j + _LG) % _NR)
                elif j - (_GC - _LG) < 2:
                    fire_g(pn, j - (_GC - _LG), (j + _LG) % _NR)

            for j in range(_GC):
                step(0, 0, 1, j, True, False)

            def group(G, carry):
                p = lax.rem(G, 2)
                pn = 1 - p
                for j in range(_GC):
                    step(G, p, pn, j, False, False)
                return carry

            lax.fori_loop(1, nfull - 1, group, 0)

            pl_ = (nfull - 1) % 2
            for j in range(_GC):
                step(nfull - 1, pl_, 1 - pl_, j, False, True)

            pt = nfull % 2
            for t in range(2):
                g = nfull * _GC + t
                b = g % _NR
                wait_s((b - _WS) % _NR)
                wait_g(b)
                fire_s(pt, t, b)
            for i in range(_WS):
                wait_s((226 - _WS + i) % _NR)

            plsc.subcore_barrier()
            pltpu.sync_copy(accum.at[pl.ds(r0, out_rows)],
                            out_hbm.at[pl.ds(r0, out_rows),
                                       pl.ds(q * _W, _W)])

        @pl.when(c == 0)
        def _():
            do_pass(0)
            do_pass(1)

        @pl.when(c == 1)
        def _():
            do_pass(2)
            do_pass(3)

    return k(tab4, zeros32, src2, dst2)


_BN = 3616


def _leaky(v):
    return jnp.maximum(v, 0.2 * v)


def _dinv(dA, dB):
    return lax.rsqrt(dA + dB + 1.0)


def _tc_conv1_table(x, sf, dA, dB, W1, n):
    def body(x_ref, sf_ref, da_ref, db_ref, w_ref, o_ref):
        dinv = _dinv(da_ref[...], db_ref[...])
        h0 = x_ref[...] * sf_ref[...]
        g1 = jnp.dot(h0, w_ref[...], preferred_element_type=jnp.float32)
        o_ref[...] = g1 * dinv

    grid = n // _BN
    bs3 = pl.BlockSpec((_BN, 3), lambda i: (i, 0))
    bs1 = pl.BlockSpec((_BN, 1), lambda i: (i, 0))
    bs128 = pl.BlockSpec((_BN, 128), lambda i: (i, 0))
    full = lambda r, c: pl.BlockSpec((r, c), lambda i: (0, 0))
    return pl.pallas_call(
        body,
        grid=(grid,),
        in_specs=[bs3, bs3, bs1, bs1, full(3, 128)],
        out_specs=bs128,
        out_shape=jax.ShapeDtypeStruct((n, 128), jnp.float32),
    )(x, sf, dA, dB, W1)


def _tc_mid(a1, t1, dA, dB, b1, W2, n):
    def body(a_ref, t_ref, da, db, b1_ref, w2_ref, o_ref):
        dinv = _dinv(da[...], db[...])
        h1 = _leaky((a_ref[...] + t_ref[...]) * dinv + b1_ref[...])
        t2 = jnp.dot(h1, w2_ref[...], preferred_element_type=jnp.float32)
        o_ref[...] = t2 * dinv

    grid = n // _BN
    bs1 = pl.BlockSpec((_BN, 1), lambda i: (i, 0))
    bs128 = pl.BlockSpec((_BN, 128), lambda i: (i, 0))
    full = lambda r, c: pl.BlockSpec((r, c), lambda i: (0, 0))
    return pl.pallas_call(
        body,
        grid=(grid,),
        in_specs=[bs128, bs128, bs1, bs1, full(1, 128), full(128, 128)],
        out_specs=bs128,
        out_shape=jax.ShapeDtypeStruct((n, 128), jnp.float32),
    )(a1, t1, dA, dB, b1, W2)


def _tc_tail(a2, t2, dA, dB, batch2, b2, gate_W, gate_b,
             p1_W, p1_b, bn_g, bn_b, p2_W, p2_b, n, num_graphs):
    grid = n // _BN
    G = num_graphs

    def body(a_ref, t_ref, da, db, bt, b2_ref, gw, gb,
             p1w, p1b, bg, bb, p2w, p2b, o_ref, zacc, dacc):
        i = pl.program_id(0)
        dinv = _dinv(da[...], db[...])
        h2 = _leaky((a_ref[...] + t_ref[...]) * dinv + b2_ref[...])
        glog = jnp.dot(h2, gw[...], preferred_element_type=jnp.float32) + gb[...]
        gate = 1.0 / (1.0 + jnp.exp(-glog))
        eg = jnp.exp(gate)
        onehot = (bt[...] == lax.broadcasted_iota(jnp.int32, (_BN, G), 1))
        onehot = onehot.astype(jnp.float32)
        zc = lax.dot_general(onehot, h2 * eg, (((0,), (0,)), ((), ())),
                             preferred_element_type=jnp.float32,
                             precision=lax.Precision.HIGHEST)
        dc = lax.dot_general(onehot, eg, (((0,), (0,)), ((), ())),
                             preferred_element_type=jnp.float32,
                             precision=lax.Precision.HIGHEST)

        @pl.when(i == 0)
        def _():
            zacc[...] = jnp.zeros_like(zacc)
            dacc[...] = jnp.zeros_like(dacc)

        zacc[...] += zc
        dacc[...] += dc

        @pl.when(i == grid - 1)
        def _():
            z = zacc[...] / (dacc[...] + 1e-16)
            z1 = jnp.dot(z, p1w[...], preferred_element_type=jnp.float32) + p1b[...]
            mu = jnp.mean(z1, axis=0, keepdims=True)
            var = jnp.mean((z1 - mu) ** 2, axis=0, keepdims=True)
            z1 = (z1 - mu) / jnp.sqrt(var + 1e-5) * bg[...] + bb[...]
            z1 = _leaky(z1)
            z2 = jnp.dot(z1, p2w[...], preferred_element_type=jnp.float32) + p2b[...]
            nrm = jnp.sqrt(jnp.sum(z2 * z2, axis=1, keepdims=True))
            o_ref[...] = z2 / jnp.maximum(nrm, 1e-12)

    bs128 = pl.BlockSpec((_BN, 128), lambda i: (i, 0))
    bs1 = pl.BlockSpec((_BN, 1), lambda i: (i, 0))
    full = lambda r, c: pl.BlockSpec((r, c), lambda i: (0, 0))
    return pl.pallas_call(
        body,
        grid=(grid,),
        in_specs=[bs128, bs128, bs1, bs1, bs1,
                  full(1, 128), full(128, 1), full(1, 1),
                  full(128, 512), full(1, 512), full(1, 512), full(1, 512),
                  full(512, 1024), full(1, 1024)],
        out_specs=full(G, 1024),
        out_shape=jax.ShapeDtypeStruct((G, 1024), jnp.float32),
        scratch_shapes=[pltpu.VMEM((G, 128), jnp.float32),
                        pltpu.VMEM((G, 1), jnp.float32)],
    )(a2, t2, dA, dB, batch2, b2, gate_W, gate_b,
      p1_W, p1_b, bn_g, bn_b, p2_W, p2_b)


def kernel(x, edge_index, batch, roi_scaler, W1, b1, W2, b2, gate_W, gate_b,
           p1_W, p1_b, bn_gamma, bn_beta, p2_W, p2_b):
    n = x.shape[0]
    e = edge_index.shape[1]
    num_graphs = n // roi_scaler.shape[0]

    src2 = jnp.pad(edge_index[0].reshape(e // _CK, _CK), ((0, _GC), (0, 0)))
    dst2 = jnp.pad(edge_index[1].reshape(e // _CK, _CK), ((0, _GC), (0, 0)))
    sf = jnp.tile(roi_scaler, (num_graphs, 1))
    ones1 = jnp.ones((_CK,), jnp.float32)
    zeros1 = jnp.zeros((n,), jnp.float32)
    batch2 = batch.reshape(n, 1)
    zeros32 = jnp.zeros((n, _W), jnp.float32)

    degA, degB = _sc_degree(dst2, ones1, zeros1, n, e)
    dA = degA.reshape(n, 1)
    dB = degB.reshape(n, 1)

    t1 = _tc_conv1_table(x, sf, dA, dB, W1, n)

    a1 = _sc_edge_stream(t1.reshape(4 * n, _W), zeros32, src2, dst2, n, e)

    t2 = _tc_mid(a1, t1, dA, dB, b1.reshape(1, 128), W2, n)

    a2 = _sc_edge_stream(t2.reshape(4 * n, _W), zeros32, src2, dst2, n, e)

    return _tc_tail(a2, t2, dA, dB, batch2,
                    b2.reshape(1, 128), gate_W, gate_b.reshape(1, 1),
                    p1_W, p1_b.reshape(1, 512), bn_gamma.reshape(1, 512),
                    bn_beta.reshape(1, 512), p2_W, p2_b.reshape(1, 1024),
                    n, num_graphs)


# speedup vs baseline: 25.2072x; 1.0441x over previous
import functools

import jax
import jax.numpy as jnp
from jax import lax
from jax.experimental import pallas as pl
from jax.experimental.pallas import tpu as pltpu
from jax.experimental.pallas import tpu_sc as plsc

_CK = 128
_GC = 16
_NR = 8
_LG = 6
_WS = _NR - _LG
_W = 32


def _sc_mesh():
    return plsc.VectorSubcoreMesh(core_axis_name="c", subcore_axis_name="s")


def _sc_degree(dst2, ones1, zeros1, n, e):
    nrows = e // _CK
    rows_per_tile = nrows // 32
    out_rows = n // 16

    @functools.partial(
        pl.kernel,
        out_type=[jax.ShapeDtypeStruct((n,), jnp.float32),
                  jax.ShapeDtypeStruct((n,), jnp.float32)],
        mesh=_sc_mesh(),
        scratch_types=[
            pltpu.VMEM((rows_per_tile, _CK), jnp.int32),
            pltpu.VMEM((_CK,), jnp.float32),
            pltpu.VMEM_SHARED((n,), jnp.float32),
            pltpu.SemaphoreType.DMA((_NR,)),
        ],
        compiler_params=pltpu.CompilerParams(use_tc_tiling_on_sc=False),
    )
    def k(dst_hbm, ones_hbm, z_hbm, outa, outb, didx, ones_v, accum, ssem):
        c = lax.axis_index("c")
        s = lax.axis_index("s")
        r0 = s * out_rows
        pltpu.sync_copy(z_hbm.at[pl.ds(r0, out_rows)],
                        accum.at[pl.ds(r0, out_rows)])
        pltpu.sync_copy(ones_hbm, ones_v)
        base = c * (nrows // 2) + s * rows_per_tile
        pltpu.sync_copy(dst_hbm.at[pl.ds(base, rows_per_tile)], didx)
        plsc.subcore_barrier()

        def fire(g, b):
            pltpu.async_copy(ones_v, accum.at[didx.at[g]], ssem.at[b],
                             add=True)

        def drain(b):
            pltpu.make_async_copy(z_hbm.at[pl.ds(0, _CK)], ones_v,
                                  ssem.at[b]).wait()

        for b in range(_NR):
            fire(b, b)

        def group(gg, carry):
            for b in range(_NR):
                drain(b)
                fire(gg * _NR + b, b)
            return carry

        lax.fori_loop(1, 14, group, 0)
        drain(0)
        fire(112, 0)
        for b in range(_NR):
            drain(b)
        plsc.subcore_barrier()

        @pl.when(c == 0)
        def _():
            pltpu.sync_copy(accum.at[pl.ds(r0, out_rows)],
                            outa.at[pl.ds(r0, out_rows)])

        @pl.when(c == 1)
        def _():
            pltpu.sync_copy(accum.at[pl.ds(r0, out_rows)],
                            outb.at[pl.ds(r0, out_rows)])

    return k(dst2, ones1, zeros1)


def _sc_edge_stream(tab4, zeros32, src2, dst2, n, e):
    nrows = e // _CK
    rows_per_tile = nrows // 16
    nfull = 14
    out_rows = n // 16

    @functools.partial(
        pl.kernel,
        out_type=jax.ShapeDtypeStruct((n, 128), jnp.float32),
        mesh=_sc_mesh(),
        scratch_types=[
            pltpu.VMEM((2, _GC, _CK), jnp.int32),
            pltpu.VMEM((2, _GC, _CK), jnp.int32),
            pltpu.VMEM((2, _GC, _CK), jnp.int32),
            pltpu.VMEM((_NR, _CK, _W), jnp.float32),
            pltpu.VMEM_SHARED((n, _W), jnp.float32),
            pltpu.SemaphoreType.DMA((2,)),
            pltpu.SemaphoreType.DMA((_NR,)),
            pltpu.SemaphoreType.DMA((_NR,)),
        ],
        compiler_params=pltpu.CompilerParams(use_tc_tiling_on_sc=False),
    )
    def k(tab4_hbm, z_hbm, src_hbm, dst_hbm, out_hbm,
          sidx, sidx4, didx, rows, accum, isem, gsem, ssem):
        c = lax.axis_index("c")
        s = lax.axis_index("s")
        r0 = s * out_rows
        base = s * rows_per_tile

        def do_pass(q):
            def fire_i(G, p):
                pltpu.async_copy(src_hbm.at[pl.ds(base + G * _GC, _GC)],
                                 sidx.at[p], isem.at[p])
                pltpu.async_copy(dst_hbm.at[pl.ds(base + G * _GC, _GC)],
                                 didx.at[p], isem.at[p])

            def wait_i(p):
                pltpu.make_async_copy(src_hbm.at[pl.ds(0, _GC)],
                                      sidx.at[p], isem.at[p]).wait()
                pltpu.make_async_copy(dst_hbm.at[pl.ds(0, _GC)],
                                      didx.at[p], isem.at[p]).wait()

            def xform(p):
                def tbody(cc, carry):
                    for kk in range(_CK // 16):
                        v = sidx[p, cc, pl.ds(kk * 16, 16)]
                        sidx4[p, cc, pl.ds(kk * 16, 16)] = v * 4 + q
                    return carry

                lax.fori_loop(0, _GC, tbody, 0)

            def fire_g(p, j, b):
                pltpu.async_copy(tab4_hbm.at[sidx4.at[p, j]], rows.at[b],
                                 gsem.at[b])

            def wait_g(b):
                pltpu.make_async_copy(tab4_hbm.at[pl.ds(0, _CK)], rows.at[b],
                                      gsem.at[b]).wait()

            def fire_s(p, j, b):
                pltpu.async_copy(rows.at[b], accum.at[didx.at[p, j]],
                                 ssem.at[b], add=True)

            def wait_s(b):
                pltpu.make_async_copy(tab4_hbm.at[pl.ds(0, _CK)], rows.at[b],
                                      ssem.at[b]).wait()

            pltpu.sync_copy(z_hbm.at[pl.ds(r0, out_rows)],
                            accum.at[pl.ds(r0, out_rows)])
            pltpu.sync_copy(src_hbm.at[pl.ds(base, _GC)], sidx.at[0])
            pltpu.sync_copy(dst_hbm.at[pl.ds(base, _GC)], didx.at[0])
            xform(0)
            for j in range(_LG):
                fire_g(0, j, j)
            plsc.subcore_barrier()

            def step(G, p, pn, j, first, last):
                b = j % _NR
                if not (first and j < _WS):
                    wait_s((j - _WS) % _NR)
                wait_g(b)
                fire_s(p, j, b)
                if j == 4:
                    fire_i(G + 1, pn)
                if j == _GC - _LG - 1:
                    wait_i(pn)
                    xform(pn)
                if j < _GC - _LG:
                    fire_g(p, j + _LG, (j + _LG) % _NR)
                elif not last:
                    fire_g(pn, j - (_GC - _LG), (j + _LG) % _NR)
                elif j - (_GC - _LG) < 2:
                    fire_g(pn, j - (_GC - _LG), (j + _LG) % _NR)

            for j in range(_GC):
                step(0, 0, 1, j, True, False)

            def group(G, carry):
                p = lax.rem(G, 2)
                pn = 1 - p
                for j in range(_GC):
                    step(G, p, pn, j, False, False)
                return carry

            lax.fori_loop(1, nfull - 1, group, 0)

            pl_ = (nfull - 1) % 2
            for j in range(_GC):
                step(nfull - 1, pl_, 1 - pl_, j, False, True)

            pt = nfull % 2
            for t in range(2):
                g = nfull * _GC + t
                b = g % _NR
                wait_s((b - _WS) % _NR)
                wait_g(b)
                fire_s(pt, t, b)
            for i in range(_WS):
                wait_s((226 - _WS + i) % _NR)

            plsc.subcore_barrier()
            pltpu.sync_copy(accum.at[pl.ds(r0, out_rows)],
                            out_hbm.at[pl.ds(r0, out_rows),
                                       pl.ds(q * _W, _W)])

        @pl.when(c == 0)
        def _():
            do_pass(0)
            do_pass(1)

        @pl.when(c == 1)
        def _():
            do_pass(2)
            do_pass(3)

    return k(tab4, zeros32, src2, dst2)


_BN = 3616


def _leaky(v):
    return jnp.maximum(v, 0.2 * v)


def _dinv(dA, dB):
    return lax.rsqrt(dA + dB + 1.0)


def _tc_conv1_table(x, sf, dA, dB, W1, n):
    def body(x_ref, sf_ref, da_ref, db_ref, w_ref, o_ref):
        dinv = _dinv(da_ref[...], db_ref[...])
        h0 = x_ref[...] * sf_ref[...]
        g1 = jnp.dot(h0, w_ref[...], preferred_element_type=jnp.float32)
        o_ref[...] = g1 * dinv

    grid = n // _BN
    bs3 = pl.BlockSpec((_BN, 3), lambda i: (i, 0))
    bs1 = pl.BlockSpec((_BN, 1), lambda i: (i, 0))
    bs128 = pl.BlockSpec((_BN, 128), lambda i: (i, 0))
    full = lambda r, c: pl.BlockSpec((r, c), lambda i: (0, 0))
    return pl.pallas_call(
        body,
        grid=(grid,),
        in_specs=[bs3, bs3, bs1, bs1, full(3, 128)],
        out_specs=bs128,
        out_shape=jax.ShapeDtypeStruct((n, 128), jnp.float32),
    )(x, sf, dA, dB, W1)


def _tc_mid(a1, t1, dA, dB, b1, W2, n):
    def body(a_ref, t_ref, da, db, b1_ref, w2_ref, o_ref):
        dinv = _dinv(da[...], db[...])
        h1 = _leaky((a_ref[...] + t_ref[...]) * dinv + b1_ref[...])
        t2 = jnp.dot(h1, w2_ref[...], preferred_element_type=jnp.float32)
        o_ref[...] = t2 * dinv

    grid = n // _BN
    bs1 = pl.BlockSpec((_BN, 1), lambda i: (i, 0))
    bs128 = pl.BlockSpec((_BN, 128), lambda i: (i, 0))
    full = lambda r, c: pl.BlockSpec((r, c), lambda i: (0, 0))
    return pl.pallas_call(
        body,
        grid=(grid,),
        in_specs=[bs128, bs128, bs1, bs1, full(1, 128), full(128, 128)],
        out_specs=bs128,
        out_shape=jax.ShapeDtypeStruct((n, 128), jnp.float32),
    )(a1, t1, dA, dB, b1, W2)


def _tc_tail(a2, t2, dA, dB, batch2, b2, gate_W, gate_b,
             p1_W, p1_b, bn_g, bn_b, p2_W, p2_b, n, num_graphs):
    grid = n // _BN
    G = num_graphs

    def body(a_ref, t_ref, da, db, bt, b2_ref, gw, gb,
             p1w, p1b, bg, bb, p2w, p2b, o_ref, zacc, dacc):
        i = pl.program_id(0)
        dinv = _dinv(da[...], db[...])
        h2 = _leaky((a_ref[...] + t_ref[...]) * dinv + b2_ref[...])
        glog = jnp.dot(h2, gw[...], preferred_element_type=jnp.float32) + gb[...]
        gate = 1.0 / (1.0 + jnp.exp(-glog))
        eg = jnp.exp(gate)
        onehot = (bt[...] == lax.broadcasted_iota(jnp.int32, (_BN, G), 1))
        onehot = onehot.astype(jnp.float32)
        zc = lax.dot_general(onehot, h2 * eg, (((0,), (0,)), ((), ())),
                             preferred_element_type=jnp.float32,
                             precision=lax.Precision.HIGHEST)
        dc = lax.dot_general(onehot, eg, (((0,), (0,)), ((), ())),
                             preferred_element_type=jnp.float32,
                             precision=lax.Precision.HIGHEST)

        @pl.when(i == 0)
        def _():
            zacc[...] = jnp.zeros_like(zacc)
            dacc[...] = jnp.zeros_like(dacc)

        zacc[...] += zc
        dacc[...] += dc

        @pl.when(i == grid - 1)
        def _():
            z = zacc[...] / (dacc[...] + 1e-16)
            z1 = jnp.dot(z, p1w[...], preferred_element_type=jnp.float32) + p1b[...]
            mu = jnp.mean(z1, axis=0, keepdims=True)
            var = jnp.mean((z1 - mu) ** 2, axis=0, keepdims=True)
            z1 = (z1 - mu) / jnp.sqrt(var + 1e-5) * bg[...] + bb[...]
            z1 = _leaky(z1)
            z2 = jnp.dot(z1, p2w[...], preferred_element_type=jnp.float32) + p2b[...]
            nrm = jnp.sqrt(jnp.sum(z2 * z2, axis=1, keepdims=True))
            o_ref[...] = z2 / jnp.maximum(nrm, 1e-12)

    bs128 = pl.BlockSpec((_BN, 128), lambda i: (i, 0))
    bs1 = pl.BlockSpec((_BN, 1), lambda i: (i, 0))
    full = lambda r, c: pl.BlockSpec((r, c), lambda i: (0, 0))
    return pl.pallas_call(
        body,
        grid=(grid,),
        in_specs=[bs128, bs128, bs1, bs1, bs1,
                  full(1, 128), full(128, 1), full(1, 1),
                  full(128, 512), full(1, 512), full(1, 512), full(1, 512),
                  full(512, 1024), full(1, 1024)],
        out_specs=full(G, 1024),
        out_shape=jax.ShapeDtypeStruct((G, 1024), jnp.float32),
        scratch_shapes=[pltpu.VMEM((G, 128), jnp.float32),
                        pltpu.VMEM((G, 1), jnp.float32)],
    )(a2, t2, dA, dB, batch2, b2, gate_W, gate_b,
      p1_W, p1_b, bn_g, bn_b, p2_W, p2_b)


def kernel(x, edge_index, batch, roi_scaler, W1, b1, W2, b2, gate_W, gate_b,
           p1_W, p1_b, bn_gamma, bn_beta, p2_W, p2_b):
    n = x.shape[0]
    e = edge_index.shape[1]
    num_graphs = n // roi_scaler.shape[0]

    src2 = jnp.pad(edge_index[0].reshape(e // _CK, _CK), ((0, _GC), (0, 0)))
    dst2 = jnp.pad(edge_index[1].reshape(e // _CK, _CK), ((0, _GC), (0, 0)))
    sf = jnp.tile(roi_scaler, (num_graphs, 1))
    ones1 = jnp.ones((_CK,), jnp.float32)
    zeros1 = jnp.zeros((n,), jnp.float32)
    batch2 = batch.reshape(n, 1)
    zeros32 = jnp.zeros((n, _W), jnp.float32)

    degA, degB = _sc_degree(dst2, ones1, zeros1, n, e)
    dA = degA.reshape(n, 1)
    dB = degB.reshape(n, 1)

    t1 = _tc_conv1_table(x, sf, dA, dB, W1, n)

    a1 = _sc_edge_stream(t1.reshape(4 * n, _W), zeros32, src2, dst2, n, e)

    t2 = _tc_mid(a1, t1, dA, dB, b1.reshape(1, 128), W2, n)

    a2 = _sc_edge_stream(t2.reshape(4 * n, _W), zeros32, src2, dst2, n, e)

    return _tc_tail(a2, t2, dA, dB, batch2,
                    b2.reshape(1, 128), gate_W, gate_b.reshape(1, 1),
                    p1_W, p1_b.reshape(1, 512), bn_gamma.reshape(1, 512),
                    bn_beta.reshape(1, 512), p2_W, p2_b.reshape(1, 1024),
                    n, num_graphs)
